# baseline (speedup 1.0000x reference)
import functools

import numpy as np
import jax
import jax.numpy as jnp
from jax import lax
from jax.experimental import pallas as pl
from jax.experimental.pallas import tpu as pltpu
from jax.experimental.pallas import tpu_sc as plsc

N = 10000
E = 160000
D = 256
H = 8

NC = 2
NS = 16
NW = NC * NS

_PERM_HM = np.array([c * 8 + h for h in range(H) for c in range(32)], np.int32)
_M_PERM = np.zeros((D, D), np.float32)
_M_PERM[np.arange(D), _PERM_HM] = 1.0


_G_CH = 160
_C = 4
_EC = E // _C


def _sc_gather(idx_hbm, table_hbm, out_hbm, idx0_v, idx1_v, rows0_v, rows1_v,
               semg, semw0, semw1):
    wid = lax.axis_index("s") * NC + lax.axis_index("c")
    nblk = 2 * _EC // _G_CH
    nper = (nblk + NW - 1) // NW
    nb_w = (nblk - wid + NW - 1) // NW
    bufs = ((idx0_v, rows0_v, semw0), (idx1_v, rows1_v, semw1))

    @pl.loop(0, (nper + 1) // 2)
    def _(q):
        for half in (0, 1):
            k = 2 * q + half
            b = wid + k * NW
            idx_v, rows_v, semw = bufs[half]

            @pl.when(b < nblk)
            def _(idx_v=idx_v, rows_v=rows_v, semw=semw, k=k, b=b):
                @pl.when(k >= 2)
                def _():
                    pltpu.make_async_copy(
                        rows_v, out_hbm.at[pl.ds(0, _G_CH)], semw).wait()

                off = b * _G_CH
                pltpu.sync_copy(idx_hbm.at[pl.ds(off, _G_CH)], idx_v)
                pltpu.async_copy(table_hbm.at[idx_v], rows_v, semg).wait()
                pltpu.async_copy(rows_v, out_hbm.at[pl.ds(off, _G_CH)], semw)

    for half in (0, 1):
        idx_v, rows_v, semw = bufs[half]

        @pl.when(nb_w >= half + 1)
        def _(rows_v=rows_v, semw=semw):
            pltpu.make_async_copy(
                rows_v, out_hbm.at[pl.ds(0, _G_CH)], semw).wait()


def _gather_rows(idx_flat, table_pk):
    mesh = plsc.VectorSubcoreMesh(core_axis_name="c", subcore_axis_name="s", num_cores=NC, num_subcores=NS)
    return pl.kernel(
        _sc_gather,
        out_type=jax.ShapeDtypeStruct((2 * _EC, D // 2), jnp.int32),
        mesh=mesh,
        scratch_types=[
            pltpu.VMEM((_G_CH,), jnp.int32),
            pltpu.VMEM((_G_CH,), jnp.int32),
            pltpu.VMEM((_G_CH, D // 2), jnp.int32),
            pltpu.VMEM((_G_CH, D // 2), jnp.int32),
            pltpu.SemaphoreType.DMA,
            pltpu.SemaphoreType.DMA,
            pltpu.SemaphoreType.DMA,
        ],
    )(idx_flat, table_pk)


_NPW = 320
_NPAD = NW * _NPW
_S_CH = 8000
_S_NCH = _EC // _S_CH
_GRP = 32
_MCAP = _S_CH + 16


def _splat_lane(vec, j):
    dnums = lax.GatherDimensionNumbers(
        offset_dims=(), collapsed_slice_dims=(0,), start_index_map=(0,))
    idx = jnp.full((16, 1), j, jnp.int32)
    return lax.gather(vec, idx, dnums, (1,),
                      mode=lax.GatherScatterMode.PROMISE_IN_BOUNDS)


def _sc_scatter_max(eidx_hbm, xx_hbm, agg_hbm, agg_v, idx_v, mid_v, mloc_v,
                    gb0_v, gb1_v, rows0_v, rows1_v, sem0, sem1):
    wid = lax.axis_index("s") * NC + lax.axis_index("c")
    base = wid * _NPW
    neg = jnp.full((16,), -jnp.inf, jnp.float32)

    @pl.loop(0, _NPW)
    def _(r):
        for k in range(D // 16):
            agg_v[r, pl.ds(k * 16, 16)] = neg

    zero16 = jnp.zeros((16,), jnp.int32)

    @pl.loop(0, _MCAP // 16)
    def _(z):
        mid_v[pl.ds(z * 16, 16)] = zero16

    @pl.loop(0, _S_NCH)
    def _(ci):
        pltpu.sync_copy(eidx_hbm.at[pl.ds(ci * _S_CH, _S_CH)], idx_v)

        def fbody(k, cnt):
            v = idx_v[pl.ds(k * 16, 16)]
            m = (v >= base) & (v < base + _NPW)
            eid = ci * _S_CH + k * 16 + lax.iota(jnp.int32, 16)
            csum = plsc.cumsum(jnp.where(m, jnp.int32(1), jnp.int32(0)))
            pos = cnt + csum - 1
            plsc.store_scatter(mid_v, [pos], eid, mask=m)
            plsc.store_scatter(mloc_v, [pos], v - base, mask=m)
            pc = plsc.all_reduce_population_count(m)
            return cnt + pc[0]

        cnt = pl.loop(0, _S_CH // 16, init_carry=jnp.int32(0), unroll=2)(fbody)
        ngrp = (cnt + (_GRP - 1)) // _GRP

        def issue(g, gb, rows, sem):
            for q in range(_GRP // 16):
                gb[pl.ds(q * 16, 16)] = mid_v[pl.ds(g * _GRP + q * 16, 16)]
            pltpu.async_copy(xx_hbm.at[gb], rows, sem)

        def wait(gb, rows, sem):
            pltpu.make_async_copy(xx_hbm.at[gb], rows, sem).wait()

        def process(g, rows):
            for q in range(_GRP // 16):
                dvec = mloc_v[pl.ds(g * _GRP + q * 16, 16)]
                for j in range(16):
                    ok = (g * _GRP + q * 16 + j) < cnt

                    @pl.when(ok)
                    def _(dvec=dvec, j=j, q=q):
                        d = jnp.minimum(jnp.maximum(dvec[j], 0), _NPW - 1)
                        r = q * 16 + j
                        for k in range(D // 16):
                            agg_v[d, pl.ds(k * 16, 16)] = jnp.maximum(
                                agg_v[d, pl.ds(k * 16, 16)],
                                rows[r, pl.ds(k * 16, 16)])

        @pl.when(ngrp > 0)
        def _():
            issue(0, gb0_v, rows0_v, sem0)

        npair = (ngrp + 1) // 2

        @pl.loop(0, npair)
        def _(p):
            g0 = 2 * p
            g1 = 2 * p + 1

            @pl.when(g1 < ngrp)
            def _():
                issue(g1, gb1_v, rows1_v, sem1)

            wait(gb0_v, rows0_v, sem0)
            process(g0, rows0_v)

            @pl.when(g1 < ngrp)
            def _():
                @pl.when(g1 + 1 < ngrp)
                def _():
                    issue(g1 + 1, gb0_v, rows0_v, sem0)

                wait(gb1_v, rows1_v, sem1)
                process(g1, rows1_v)

    pltpu.sync_copy(agg_v, agg_hbm.at[pl.ds(base, _NPW)])


def _scatter_max(eidx_flat, xx):
    mesh = plsc.VectorSubcoreMesh(core_axis_name="c", subcore_axis_name="s", num_cores=NC, num_subcores=NS)
    return pl.kernel(
        functools.partial(_sc_scatter_max),
        out_type=jax.ShapeDtypeStruct((_NPAD, D), jnp.float32),
        mesh=mesh,
        compiler_params=pltpu.CompilerParams(needs_layout_passes=False),
        scratch_types=[
            pltpu.VMEM((_NPW, D), jnp.float32),
            pltpu.VMEM((_S_CH,), jnp.int32),
            pltpu.VMEM((_MCAP,), jnp.int32),
            pltpu.VMEM((_MCAP,), jnp.int32),
            pltpu.VMEM((_GRP,), jnp.int32),
            pltpu.VMEM((_GRP,), jnp.int32),
            pltpu.VMEM((_GRP, D), jnp.float32),
            pltpu.VMEM((_GRP, D), jnp.float32),
            pltpu.SemaphoreType.DMA,
            pltpu.SemaphoreType.DMA,
        ],
    )(eidx_flat, xx)


_BE = 800


def _edge_body(gi_ref, gj_ref, ef_ref, w1a, w1b, w1c, b1, w2, b2,
               aw, bw, c0, wv, bvr, w2dr, b2dr, sblk, mp,
               gcn_ref, probo_ref, xx_ref):
    gi = gi_ref[...].astype(jnp.float32)
    gj = gj_ref[...].astype(jnp.float32)
    ef = ef_ref[...]
    hh = jnp.maximum(gi @ w1a[...] + ef @ w1b[...] + gj @ w1c[...] + b1[...], 0.0)
    gcn_ref[...] = hh @ w2[...] + b2[...]
    t1 = jnp.maximum(gi @ aw[...] + ef @ bw[...] + c0[...], 0.0)
    p2 = t1 @ w2dr[...] + b2dr[...]
    c = jnp.max(p2, axis=1, keepdims=True)
    ex = jnp.exp(p2 - c)
    ssum = ex @ sblk[...]
    prob_hm = ex / ssum
    probo_ref[...] = prob_hm @ mp[...]
    xx_ref[...] = prob_hm * (gj @ wv[...] + bvr[...])


def _edge_compute(G, ef, w1a, w1b, w1c, b1, w2, b2, aw, bw, c0,
                  wv, bv_hm, w2d, b2d, sblk, mperm):
    nblk = _EC // _BE
    row_spec = lambda off: pl.BlockSpec((_BE, D), lambda i, o=off: (i + o, 0))
    full = lambda a: pl.BlockSpec(a.shape, lambda i: (0,) * a.ndim)
    weights = (w1a, w1b, w1c, b1, w2, b2, aw, bw, c0, wv, bv_hm,
               w2d, b2d, sblk, mperm)
    return pl.pallas_call(
        _edge_body,
        grid=(nblk,),
        in_specs=[row_spec(0), row_spec(nblk), row_spec(0)]
        + [full(w) for w in weights],
        out_specs=[pl.BlockSpec((_BE, D), lambda i: (i, 0))] * 3,
        out_shape=[jax.ShapeDtypeStruct((_EC, D), jnp.float32)] * 3,
    )(G, G, ef, *weights)


_BN = 400


def _node_body(x_ref, a0_ref, a1_ref, a2_ref, a3_ref, w1t, w1b, b1, w2, b2,
               out_ref):
    a = jnp.maximum(jnp.maximum(a0_ref[...], a1_ref[...]),
                    jnp.maximum(a2_ref[...], a3_ref[...]))
    a = jnp.where(a == -jnp.inf, 0.0, a)
    h2 = jnp.maximum(x_ref[...] @ w1t[...] + a @ w1b[...] + b1[...], 0.0)
    out_ref[...] = h2 @ w2[...] + b2[...]


def _node_compute(x, aggs, w1t, w1b, b1, w2, b2):
    full = lambda a: pl.BlockSpec(a.shape, lambda i: (0,) * a.ndim)
    weights = (w1t, w1b, b1, w2, b2)
    return pl.pallas_call(
        _node_body,
        grid=(N // _BN,),
        in_specs=[pl.BlockSpec((_BN, D), lambda i: (i, 0))] * 5
        + [full(w) for w in weights],
        out_specs=pl.BlockSpec((_BN, D), lambda i: (i, 0)),
        out_shape=jax.ShapeDtypeStruct((N, D), jnp.float32),
    )(x, *aggs, *weights)


def kernel(x, edge_feature, edge_index, ne_W1, ne_b1, ne_W2, ne_b2, Wq, bq,
           We, be, Wv, bv, nn_W1, nn_b1, nn_W2, nn_b2, pr_W1, pr_b1, pr_W2,
           pr_b2):
    perm = jnp.asarray(_PERM_HM)
    eye8 = jnp.eye(H, dtype=jnp.float32)

    w1a = ne_W1[0:D]
    w1b = ne_W1[D:2 * D]
    w1c = ne_W1[2 * D:3 * D]

    wq_hm = Wq[:, perm]
    bq_hm = bq[perm]
    we_hm = We[:, perm]
    be_hm = be[perm]
    wv_hm = Wv[:, perm]
    bv_hm = bv[perm]

    w1t = jnp.kron(eye8, nn_W1[:32, :])
    w1btm = jnp.kron(eye8, nn_W1[32:, :])
    w2d = jnp.kron(eye8, nn_W2)
    b1d = jnp.tile(nn_b1, H)
    b2d = jnp.tile(nn_b2, H)
    aw = wq_hm @ w1t
    bw = we_hm @ w1btm
    c0 = bq_hm @ w1t + be_hm @ w1btm + b1d
    sblk = jnp.kron(eye8, jnp.ones((32, 32), jnp.float32))
    mperm = jnp.asarray(_M_PERM)

    pr_w1t = pr_W1[:D]
    pr_w1b = pr_W1[D:][perm]

    row2 = lambda a: a.reshape(1, -1)

    idx_i = edge_index[0]
    idx_j = edge_index[1]
    x_pk = jax.lax.bitcast_convert_type(
        x.astype(jnp.bfloat16).reshape(N, D // 2, 2), jnp.int32)
    gcns, probs, aggs = [], [], []
    for c in range(_C):
        sl = slice(c * _EC, (c + 1) * _EC)
        idx_cat = jnp.concatenate([idx_i[sl], idx_j[sl]])
        Gc_pk = _gather_rows(idx_cat, x_pk)
        Gc = jax.lax.bitcast_convert_type(Gc_pk, jnp.bfloat16).reshape(
            2 * _EC, D)
        gcn_c, probO_c, xx_c = _edge_compute(
            Gc, edge_feature[sl], w1a, w1b, w1c, row2(ne_b1), ne_W2,
            row2(ne_b2), aw, bw, row2(c0), wv_hm, row2(bv_hm), w2d,
            row2(b2d), sblk, mperm)
        agg_c = _scatter_max(idx_i[sl], xx_c)
        gcns.append(gcn_c)
        probs.append(probO_c)
        aggs.append(agg_c)

    out = _node_compute(x, aggs, pr_w1t, pr_w1b, row2(pr_b1), pr_W2,
                        row2(pr_b2))

    gcn = jnp.concatenate(gcns, axis=0)
    probO = jnp.concatenate(probs, axis=0)
    return (out, gcn, probO.reshape(E, 32, H))


# speedup vs baseline: 1.9518x; 1.9518x over previous
import functools

import numpy as np
import jax
import jax.numpy as jnp
from jax import lax
from jax.experimental import pallas as pl
from jax.experimental.pallas import tpu as pltpu
from jax.experimental.pallas import tpu_sc as plsc

N = 10000
E = 160000
D = 256
H = 8

NC = 2
NS = 16
NW = NC * NS

_PERM_HM = np.array([c * 8 + h for h in range(H) for c in range(32)], np.int32)
_M_PERM = np.zeros((D, D), np.float32)
_M_PERM[np.arange(D), _PERM_HM] = 1.0


_G_CH = 160
_C = 4
_EC = E // _C


def _sc_gather(idx_hbm, table_hbm, out_hbm, idx0_v, idx1_v, rows0_v, rows1_v,
               semg, semw0, semw1):
    wid = lax.axis_index("s") * NC + lax.axis_index("c")
    nblk = 2 * _EC // _G_CH
    nper = (nblk + NW - 1) // NW
    nb_w = (nblk - wid + NW - 1) // NW
    bufs = ((idx0_v, rows0_v, semw0), (idx1_v, rows1_v, semw1))

    @pl.loop(0, (nper + 1) // 2)
    def _(q):
        for half in (0, 1):
            k = 2 * q + half
            b = wid + k * NW
            idx_v, rows_v, semw = bufs[half]

            @pl.when(b < nblk)
            def _(idx_v=idx_v, rows_v=rows_v, semw=semw, k=k, b=b):
                @pl.when(k >= 2)
                def _():
                    pltpu.make_async_copy(
                        rows_v, out_hbm.at[pl.ds(0, _G_CH)], semw).wait()

                off = b * _G_CH
                pltpu.sync_copy(idx_hbm.at[pl.ds(off, _G_CH)], idx_v)
                pltpu.async_copy(table_hbm.at[idx_v], rows_v, semg).wait()
                pltpu.async_copy(rows_v, out_hbm.at[pl.ds(off, _G_CH)], semw)

    for half in (0, 1):
        idx_v, rows_v, semw = bufs[half]

        @pl.when(nb_w >= half + 1)
        def _(rows_v=rows_v, semw=semw):
            pltpu.make_async_copy(
                rows_v, out_hbm.at[pl.ds(0, _G_CH)], semw).wait()


def _gather_rows(idx_flat, table):
    mesh = plsc.VectorSubcoreMesh(core_axis_name="c", subcore_axis_name="s", num_cores=NC, num_subcores=NS)
    return pl.kernel(
        _sc_gather,
        out_type=jax.ShapeDtypeStruct((2 * _EC, D), jnp.float32),
        mesh=mesh,
        scratch_types=[
            pltpu.VMEM((_G_CH,), jnp.int32),
            pltpu.VMEM((_G_CH,), jnp.int32),
            pltpu.VMEM((_G_CH, D), jnp.float32),
            pltpu.VMEM((_G_CH, D), jnp.float32),
            pltpu.SemaphoreType.DMA,
            pltpu.SemaphoreType.DMA,
            pltpu.SemaphoreType.DMA,
        ],
    )(idx_flat, table)


_NPW = 320
_NPAD = NW * _NPW
_S_CH = 8000
_S_NCH = _EC // _S_CH
_GRP = 32
_MCAP = _S_CH + 16


def _splat_lane(vec, j):
    dnums = lax.GatherDimensionNumbers(
        offset_dims=(), collapsed_slice_dims=(0,), start_index_map=(0,))
    idx = jnp.full((16, 1), j, jnp.int32)
    return lax.gather(vec, idx, dnums, (1,),
                      mode=lax.GatherScatterMode.PROMISE_IN_BOUNDS)


def _sc_scatter_max(eidx_hbm, xx_hbm, agg_hbm, agg_v, idx_v, mid_v, mloc_v,
                    gb0_v, gb1_v, rows0_v, rows1_v, sem0, sem1):
    wid = lax.axis_index("s") * NC + lax.axis_index("c")
    base = wid * _NPW
    neg = jnp.full((16,), -jnp.inf, jnp.float32)

    @pl.loop(0, _NPW)
    def _(r):
        for k in range(D // 16):
            agg_v[r, pl.ds(k * 16, 16)] = neg

    zero16 = jnp.zeros((16,), jnp.int32)

    @pl.loop(0, _MCAP // 16)
    def _(z):
        mid_v[pl.ds(z * 16, 16)] = zero16

    @pl.loop(0, _S_NCH)
    def _(ci):
        pltpu.sync_copy(eidx_hbm.at[pl.ds(ci * _S_CH, _S_CH)], idx_v)

        def fbody(k, cnt):
            v = idx_v[pl.ds(k * 16, 16)]
            m = (v >= base) & (v < base + _NPW)
            eid = ci * _S_CH + k * 16 + lax.iota(jnp.int32, 16)
            csum = plsc.cumsum(jnp.where(m, jnp.int32(1), jnp.int32(0)))
            pos = cnt + csum - 1
            plsc.store_scatter(mid_v, [pos], eid, mask=m)
            plsc.store_scatter(mloc_v, [pos], v - base, mask=m)
            pc = plsc.all_reduce_population_count(m)
            return cnt + pc[0]

        cnt = pl.loop(0, _S_CH // 16, init_carry=jnp.int32(0), unroll=2)(fbody)
        ngrp = (cnt + (_GRP - 1)) // _GRP

        def issue(g, gb, rows, sem):
            for q in range(_GRP // 16):
                gb[pl.ds(q * 16, 16)] = mid_v[pl.ds(g * _GRP + q * 16, 16)]
            pltpu.async_copy(xx_hbm.at[gb], rows, sem)

        def wait(gb, rows, sem):
            pltpu.make_async_copy(xx_hbm.at[gb], rows, sem).wait()

        def process(g, rows):
            for q in range(_GRP // 16):
                dvec = mloc_v[pl.ds(g * _GRP + q * 16, 16)]
                for j in range(16):
                    ok = (g * _GRP + q * 16 + j) < cnt

                    @pl.when(ok)
                    def _(dvec=dvec, j=j, q=q):
                        d = jnp.minimum(jnp.maximum(dvec[j], 0), _NPW - 1)
                        r = q * 16 + j
                        for k in range(D // 16):
                            agg_v[d, pl.ds(k * 16, 16)] = jnp.maximum(
                                agg_v[d, pl.ds(k * 16, 16)],
                                rows[r, pl.ds(k * 16, 16)])

        @pl.when(ngrp > 0)
        def _():
            issue(0, gb0_v, rows0_v, sem0)

        npair = (ngrp + 1) // 2

        @pl.loop(0, npair)
        def _(p):
            g0 = 2 * p
            g1 = 2 * p + 1

            @pl.when(g1 < ngrp)
            def _():
                issue(g1, gb1_v, rows1_v, sem1)

            wait(gb0_v, rows0_v, sem0)
            process(g0, rows0_v)

            @pl.when(g1 < ngrp)
            def _():
                @pl.when(g1 + 1 < ngrp)
                def _():
                    issue(g1 + 1, gb0_v, rows0_v, sem0)

                wait(gb1_v, rows1_v, sem1)
                process(g1, rows1_v)

    pltpu.sync_copy(agg_v, agg_hbm.at[pl.ds(base, _NPW)])


def _scatter_max(eidx_flat, xx):
    mesh = plsc.VectorSubcoreMesh(core_axis_name="c", subcore_axis_name="s", num_cores=NC, num_subcores=NS)
    return pl.kernel(
        functools.partial(_sc_scatter_max),
        out_type=jax.ShapeDtypeStruct((_NPAD, D), jnp.float32),
        mesh=mesh,
        compiler_params=pltpu.CompilerParams(needs_layout_passes=False),
        scratch_types=[
            pltpu.VMEM((_NPW, D), jnp.float32),
            pltpu.VMEM((_S_CH,), jnp.int32),
            pltpu.VMEM((_MCAP,), jnp.int32),
            pltpu.VMEM((_MCAP,), jnp.int32),
            pltpu.VMEM((_GRP,), jnp.int32),
            pltpu.VMEM((_GRP,), jnp.int32),
            pltpu.VMEM((_GRP, D), jnp.float32),
            pltpu.VMEM((_GRP, D), jnp.float32),
            pltpu.SemaphoreType.DMA,
            pltpu.SemaphoreType.DMA,
        ],
    )(eidx_flat, xx)


_BE = 800


def _edge_body(gi_ref, gj_ref, ef_ref, w1a, w1b, w1c, b1, w2, b2,
               aw, bw, c0, wv, bvr, w2dr, b2dr, sblk, mp,
               gcn_ref, probo_ref, xx_ref):
    gi = gi_ref[...]
    gj = gj_ref[...]
    ef = ef_ref[...]
    hh = jnp.maximum(gi @ w1a[...] + ef @ w1b[...] + gj @ w1c[...] + b1[...], 0.0)
    gcn_ref[...] = hh @ w2[...] + b2[...]
    t1 = jnp.maximum(gi @ aw[...] + ef @ bw[...] + c0[...], 0.0)
    p2 = t1 @ w2dr[...] + b2dr[...]
    c = jnp.max(p2, axis=1, keepdims=True)
    ex = jnp.exp(p2 - c)
    ssum = ex @ sblk[...]
    prob_hm = ex / ssum
    probo_ref[...] = prob_hm @ mp[...]
    xx_ref[...] = prob_hm * (gj @ wv[...] + bvr[...])


def _edge_compute(G, ef, w1a, w1b, w1c, b1, w2, b2, aw, bw, c0,
                  wv, bv_hm, w2d, b2d, sblk, mperm):
    nblk = _EC // _BE
    row_spec = lambda off: pl.BlockSpec((_BE, D), lambda i, o=off: (i + o, 0))
    full = lambda a: pl.BlockSpec(a.shape, lambda i: (0,) * a.ndim)
    weights = (w1a, w1b, w1c, b1, w2, b2, aw, bw, c0, wv, bv_hm,
               w2d, b2d, sblk, mperm)
    return pl.pallas_call(
        _edge_body,
        grid=(nblk,),
        in_specs=[row_spec(0), row_spec(nblk), row_spec(0)]
        + [full(w) for w in weights],
        out_specs=[pl.BlockSpec((_BE, D), lambda i: (i, 0))] * 3,
        out_shape=[jax.ShapeDtypeStruct((_EC, D), jnp.float32)] * 3,
    )(G, G, ef, *weights)


_BN = 400


def _node_body(x_ref, a0_ref, a1_ref, a2_ref, a3_ref, w1t, w1b, b1, w2, b2,
               out_ref):
    a = jnp.maximum(jnp.maximum(a0_ref[...], a1_ref[...]),
                    jnp.maximum(a2_ref[...], a3_ref[...]))
    a = jnp.where(a == -jnp.inf, 0.0, a)
    h2 = jnp.maximum(x_ref[...] @ w1t[...] + a @ w1b[...] + b1[...], 0.0)
    out_ref[...] = h2 @ w2[...] + b2[...]


def _node_compute(x, aggs, w1t, w1b, b1, w2, b2):
    full = lambda a: pl.BlockSpec(a.shape, lambda i: (0,) * a.ndim)
    weights = (w1t, w1b, b1, w2, b2)
    return pl.pallas_call(
        _node_body,
        grid=(N // _BN,),
        in_specs=[pl.BlockSpec((_BN, D), lambda i: (i, 0))] * 5
        + [full(w) for w in weights],
        out_specs=pl.BlockSpec((_BN, D), lambda i: (i, 0)),
        out_shape=jax.ShapeDtypeStruct((N, D), jnp.float32),
    )(x, *aggs, *weights)


def kernel(x, edge_feature, edge_index, ne_W1, ne_b1, ne_W2, ne_b2, Wq, bq,
           We, be, Wv, bv, nn_W1, nn_b1, nn_W2, nn_b2, pr_W1, pr_b1, pr_W2,
           pr_b2):
    perm = jnp.asarray(_PERM_HM)
    eye8 = jnp.eye(H, dtype=jnp.float32)

    w1a = ne_W1[0:D]
    w1b = ne_W1[D:2 * D]
    w1c = ne_W1[2 * D:3 * D]

    wq_hm = Wq[:, perm]
    bq_hm = bq[perm]
    we_hm = We[:, perm]
    be_hm = be[perm]
    wv_hm = Wv[:, perm]
    bv_hm = bv[perm]

    w1t = jnp.kron(eye8, nn_W1[:32, :])
    w1btm = jnp.kron(eye8, nn_W1[32:, :])
    w2d = jnp.kron(eye8, nn_W2)
    b1d = jnp.tile(nn_b1, H)
    b2d = jnp.tile(nn_b2, H)
    aw = wq_hm @ w1t
    bw = we_hm @ w1btm
    c0 = bq_hm @ w1t + be_hm @ w1btm + b1d
    sblk = jnp.kron(eye8, jnp.ones((32, 32), jnp.float32))
    mperm = jnp.asarray(_M_PERM)

    pr_w1t = pr_W1[:D]
    pr_w1b = pr_W1[D:][perm]

    row2 = lambda a: a.reshape(1, -1)

    idx_i = edge_index[0]
    idx_j = edge_index[1]
    gcns, probs, aggs = [], [], []
    for c in range(_C):
        sl = slice(c * _EC, (c + 1) * _EC)
        idx_cat = jnp.concatenate([idx_i[sl], idx_j[sl]])
        Gc = _gather_rows(idx_cat, x)
        gcn_c, probO_c, xx_c = _edge_compute(
            Gc, edge_feature[sl], w1a, w1b, w1c, row2(ne_b1), ne_W2,
            row2(ne_b2), aw, bw, row2(c0), wv_hm, row2(bv_hm), w2d,
            row2(b2d), sblk, mperm)
        agg_c = _scatter_max(idx_i[sl], xx_c)
        gcns.append(gcn_c)
        probs.append(probO_c)
        aggs.append(agg_c)

    out = _node_compute(x, aggs, pr_w1t, pr_w1b, row2(pr_b1), pr_W2,
                        row2(pr_b2))

    gcn = jnp.concatenate(gcns, axis=0)
    probO = jnp.concatenate(probs, axis=0)
    return (out, gcn, probO.reshape(E, 32, H))


# speedup vs baseline: 1.9765x; 1.0127x over previous
import functools

import numpy as np
import jax
import jax.numpy as jnp
from jax import lax
from jax.experimental import pallas as pl
from jax.experimental.pallas import tpu as pltpu
from jax.experimental.pallas import tpu_sc as plsc

N = 10000
E = 160000
D = 256
H = 8

NC = 2
NS = 16
NW = NC * NS

_PERM_HM = np.array([c * 8 + h for h in range(H) for c in range(32)], np.int32)
_M_PERM = np.zeros((D, D), np.float32)
_M_PERM[np.arange(D), _PERM_HM] = 1.0


_G_CH = 160
_C = 4
_EC = E // _C


def _sc_gather(idx_hbm, table_hbm, out_hbm, idx0_v, idx1_v, rows0_v, rows1_v,
               semg, semw0, semw1):
    wid = lax.axis_index("s") * NC + lax.axis_index("c")
    nblk = 2 * _EC // _G_CH
    nper = (nblk + NW - 1) // NW
    nb_w = (nblk - wid + NW - 1) // NW
    bufs = ((idx0_v, rows0_v, semw0), (idx1_v, rows1_v, semw1))

    @pl.loop(0, (nper + 1) // 2)
    def _(q):
        for half in (0, 1):
            k = 2 * q + half
            b = wid + k * NW
            idx_v, rows_v, semw = bufs[half]

            @pl.when(b < nblk)
            def _(idx_v=idx_v, rows_v=rows_v, semw=semw, k=k, b=b):
                @pl.when(k >= 2)
                def _():
                    pltpu.make_async_copy(
                        rows_v, out_hbm.at[pl.ds(0, _G_CH)], semw).wait()

                off = b * _G_CH
                pltpu.sync_copy(idx_hbm.at[pl.ds(off, _G_CH)], idx_v)
                pltpu.async_copy(table_hbm.at[idx_v], rows_v, semg).wait()
                pltpu.async_copy(rows_v, out_hbm.at[pl.ds(off, _G_CH)], semw)

    for half in (0, 1):
        idx_v, rows_v, semw = bufs[half]

        @pl.when(nb_w >= half + 1)
        def _(rows_v=rows_v, semw=semw):
            pltpu.make_async_copy(
                rows_v, out_hbm.at[pl.ds(0, _G_CH)], semw).wait()


def _gather_rows(idx_flat, table):
    mesh = plsc.VectorSubcoreMesh(core_axis_name="c", subcore_axis_name="s", num_cores=NC, num_subcores=NS)
    return pl.kernel(
        _sc_gather,
        out_type=jax.ShapeDtypeStruct((2 * _EC, D), jnp.float32),
        mesh=mesh,
        scratch_types=[
            pltpu.VMEM((_G_CH,), jnp.int32),
            pltpu.VMEM((_G_CH,), jnp.int32),
            pltpu.VMEM((_G_CH, D), jnp.float32),
            pltpu.VMEM((_G_CH, D), jnp.float32),
            pltpu.SemaphoreType.DMA,
            pltpu.SemaphoreType.DMA,
            pltpu.SemaphoreType.DMA,
        ],
    )(idx_flat, table)


_NPW = 320
_NPAD = NW * _NPW
_S_CH = 10000
_S_NCH = _EC // _S_CH
_GRP = 32
_MCAP = _S_CH + 16


def _splat_lane(vec, j):
    dnums = lax.GatherDimensionNumbers(
        offset_dims=(), collapsed_slice_dims=(0,), start_index_map=(0,))
    idx = jnp.full((16, 1), j, jnp.int32)
    return lax.gather(vec, idx, dnums, (1,),
                      mode=lax.GatherScatterMode.PROMISE_IN_BOUNDS)


def _sc_scatter_max(eidx_hbm, xx_hbm, agg_hbm, agg_v, idx_v, mid_v, mloc_v,
                    gb0_v, gb1_v, rows0_v, rows1_v, sem0, sem1):
    wid = lax.axis_index("s") * NC + lax.axis_index("c")
    base = wid * _NPW
    neg = jnp.full((16,), -jnp.inf, jnp.float32)

    @pl.loop(0, _NPW)
    def _(r):
        for k in range(D // 16):
            agg_v[r, pl.ds(k * 16, 16)] = neg

    zero16 = jnp.zeros((16,), jnp.int32)

    @pl.loop(0, _MCAP // 16)
    def _(z):
        mid_v[pl.ds(z * 16, 16)] = zero16

    @pl.loop(0, _S_NCH)
    def _(ci):
        pltpu.sync_copy(eidx_hbm.at[pl.ds(ci * _S_CH, _S_CH)], idx_v)

        def fbody(k, cnt):
            v = idx_v[pl.ds(k * 16, 16)]
            m = (v >= base) & (v < base + _NPW)
            eid = ci * _S_CH + k * 16 + lax.iota(jnp.int32, 16)
            csum = plsc.cumsum(jnp.where(m, jnp.int32(1), jnp.int32(0)))
            pos = cnt + csum - 1
            plsc.store_scatter(mid_v, [pos], eid, mask=m)
            plsc.store_scatter(mloc_v, [pos], v - base, mask=m)
            pc = plsc.all_reduce_population_count(m)
            return cnt + pc[0]

        cnt = pl.loop(0, _S_CH // 16, init_carry=jnp.int32(0), unroll=2)(fbody)
        ngrp = (cnt + (_GRP - 1)) // _GRP

        def issue(g, gb, rows, sem):
            for q in range(_GRP // 16):
                gb[pl.ds(q * 16, 16)] = mid_v[pl.ds(g * _GRP + q * 16, 16)]
            pltpu.async_copy(xx_hbm.at[gb], rows, sem)

        def wait(gb, rows, sem):
            pltpu.make_async_copy(xx_hbm.at[gb], rows, sem).wait()

        def process(g, rows):
            for q in range(_GRP // 16):
                dvec = mloc_v[pl.ds(g * _GRP + q * 16, 16)]
                for j in range(16):
                    ok = (g * _GRP + q * 16 + j) < cnt

                    @pl.when(ok)
                    def _(dvec=dvec, j=j, q=q):
                        d = jnp.minimum(jnp.maximum(dvec[j], 0), _NPW - 1)
                        r = q * 16 + j
                        for k in range(D // 16):
                            agg_v[d, pl.ds(k * 16, 16)] = jnp.maximum(
                                agg_v[d, pl.ds(k * 16, 16)],
                                rows[r, pl.ds(k * 16, 16)])

        @pl.when(ngrp > 0)
        def _():
            issue(0, gb0_v, rows0_v, sem0)

        npair = (ngrp + 1) // 2

        @pl.loop(0, npair)
        def _(p):
            g0 = 2 * p
            g1 = 2 * p + 1

            @pl.when(g1 < ngrp)
            def _():
                issue(g1, gb1_v, rows1_v, sem1)

            wait(gb0_v, rows0_v, sem0)
            process(g0, rows0_v)

            @pl.when(g1 < ngrp)
            def _():
                @pl.when(g1 + 1 < ngrp)
                def _():
                    issue(g1 + 1, gb0_v, rows0_v, sem0)

                wait(gb1_v, rows1_v, sem1)
                process(g1, rows1_v)

    pltpu.sync_copy(agg_v, agg_hbm.at[pl.ds(base, _NPW)])


def _scatter_max(eidx_flat, xx):
    mesh = plsc.VectorSubcoreMesh(core_axis_name="c", subcore_axis_name="s", num_cores=NC, num_subcores=NS)
    return pl.kernel(
        functools.partial(_sc_scatter_max),
        out_type=jax.ShapeDtypeStruct((_NPAD, D), jnp.float32),
        mesh=mesh,
        compiler_params=pltpu.CompilerParams(needs_layout_passes=False),
        scratch_types=[
            pltpu.VMEM((_NPW, D), jnp.float32),
            pltpu.VMEM((_S_CH,), jnp.int32),
            pltpu.VMEM((_MCAP,), jnp.int32),
            pltpu.VMEM((_MCAP,), jnp.int32),
            pltpu.VMEM((_GRP,), jnp.int32),
            pltpu.VMEM((_GRP,), jnp.int32),
            pltpu.VMEM((_GRP, D), jnp.float32),
            pltpu.VMEM((_GRP, D), jnp.float32),
            pltpu.SemaphoreType.DMA,
            pltpu.SemaphoreType.DMA,
        ],
    )(eidx_flat, xx)


_BE = 800


def _edge_body(gi_ref, gj_ref, ef_ref, w1a, w1b, w1c, b1, w2, b2,
               aw, bw, c0, wv, bvr, w2dr, b2dr, sblk, mp,
               gcn_ref, probo_ref, xx_ref):
    gi = gi_ref[...]
    gj = gj_ref[...]
    ef = ef_ref[...]
    hh = jnp.maximum(gi @ w1a[...] + ef @ w1b[...] + gj @ w1c[...] + b1[...], 0.0)
    gcn_ref[...] = hh @ w2[...] + b2[...]
    t1 = jnp.maximum(gi @ aw[...] + ef @ bw[...] + c0[...], 0.0)
    p2 = t1 @ w2dr[...] + b2dr[...]
    c = jnp.max(p2, axis=1, keepdims=True)
    ex = jnp.exp(p2 - c)
    ssum = ex @ sblk[...]
    prob_hm = ex / ssum
    probo_ref[...] = prob_hm @ mp[...]
    xx_ref[...] = prob_hm * (gj @ wv[...] + bvr[...])


def _edge_compute(G, ef, w1a, w1b, w1c, b1, w2, b2, aw, bw, c0,
                  wv, bv_hm, w2d, b2d, sblk, mperm):
    nblk = _EC // _BE
    row_spec = lambda off: pl.BlockSpec((_BE, D), lambda i, o=off: (i + o, 0))
    full = lambda a: pl.BlockSpec(a.shape, lambda i: (0,) * a.ndim)
    weights = (w1a, w1b, w1c, b1, w2, b2, aw, bw, c0, wv, bv_hm,
               w2d, b2d, sblk, mperm)
    return pl.pallas_call(
        _edge_body,
        grid=(nblk,),
        in_specs=[row_spec(0), row_spec(nblk), row_spec(0)]
        + [full(w) for w in weights],
        out_specs=[pl.BlockSpec((_BE, D), lambda i: (i, 0))] * 3,
        out_shape=[jax.ShapeDtypeStruct((_EC, D), jnp.float32)] * 3,
    )(G, G, ef, *weights)


_BN = 400


def _node_body(x_ref, a0_ref, a1_ref, a2_ref, a3_ref, w1t, w1b, b1, w2, b2,
               out_ref):
    a = jnp.maximum(jnp.maximum(a0_ref[...], a1_ref[...]),
                    jnp.maximum(a2_ref[...], a3_ref[...]))
    a = jnp.where(a == -jnp.inf, 0.0, a)
    h2 = jnp.maximum(x_ref[...] @ w1t[...] + a @ w1b[...] + b1[...], 0.0)
    out_ref[...] = h2 @ w2[...] + b2[...]


def _node_compute(x, aggs, w1t, w1b, b1, w2, b2):
    full = lambda a: pl.BlockSpec(a.shape, lambda i: (0,) * a.ndim)
    weights = (w1t, w1b, b1, w2, b2)
    return pl.pallas_call(
        _node_body,
        grid=(N // _BN,),
        in_specs=[pl.BlockSpec((_BN, D), lambda i: (i, 0))] * 5
        + [full(w) for w in weights],
        out_specs=pl.BlockSpec((_BN, D), lambda i: (i, 0)),
        out_shape=jax.ShapeDtypeStruct((N, D), jnp.float32),
    )(x, *aggs, *weights)


def kernel(x, edge_feature, edge_index, ne_W1, ne_b1, ne_W2, ne_b2, Wq, bq,
           We, be, Wv, bv, nn_W1, nn_b1, nn_W2, nn_b2, pr_W1, pr_b1, pr_W2,
           pr_b2):
    perm = jnp.asarray(_PERM_HM)
    eye8 = jnp.eye(H, dtype=jnp.float32)

    w1a = ne_W1[0:D]
    w1b = ne_W1[D:2 * D]
    w1c = ne_W1[2 * D:3 * D]

    wq_hm = Wq[:, perm]
    bq_hm = bq[perm]
    we_hm = We[:, perm]
    be_hm = be[perm]
    wv_hm = Wv[:, perm]
    bv_hm = bv[perm]

    w1t = jnp.kron(eye8, nn_W1[:32, :])
    w1btm = jnp.kron(eye8, nn_W1[32:, :])
    w2d = jnp.kron(eye8, nn_W2)
    b1d = jnp.tile(nn_b1, H)
    b2d = jnp.tile(nn_b2, H)
    aw = wq_hm @ w1t
    bw = we_hm @ w1btm
    c0 = bq_hm @ w1t + be_hm @ w1btm + b1d
    sblk = jnp.kron(eye8, jnp.ones((32, 32), jnp.float32))
    mperm = jnp.asarray(_M_PERM)

    pr_w1t = pr_W1[:D]
    pr_w1b = pr_W1[D:][perm]

    row2 = lambda a: a.reshape(1, -1)

    idx_i = edge_index[0]
    idx_j = edge_index[1]
    gcns, probs, aggs = [], [], []
    for c in range(_C):
        sl = slice(c * _EC, (c + 1) * _EC)
        idx_cat = jnp.concatenate([idx_i[sl], idx_j[sl]])
        Gc = _gather_rows(idx_cat, x)
        gcn_c, probO_c, xx_c = _edge_compute(
            Gc, edge_feature[sl], w1a, w1b, w1c, row2(ne_b1), ne_W2,
            row2(ne_b2), aw, bw, row2(c0), wv_hm, row2(bv_hm), w2d,
            row2(b2d), sblk, mperm)
        agg_c = _scatter_max(idx_i[sl], xx_c)
        gcns.append(gcn_c)
        probs.append(probO_c)
        aggs.append(agg_c)

    out = _node_compute(x, aggs, pr_w1t, pr_w1b, row2(pr_b1), pr_W2,
                        row2(pr_b2))

    gcn = jnp.concatenate(gcns, axis=0)
    probO = jnp.concatenate(probs, axis=0)
    return (out, gcn, probO.reshape(E, 32, H))
